# Initial kernel scaffold; baseline (speedup 1.0000x reference)
import jax
import jax.numpy as jnp
from jax.experimental import pallas as pl


def kernel(x, edge_index, W_l, W_r):
    raise NotImplementedError("write your pallas kernel here")


# speedup vs baseline: 8.2129x; 8.2129x over previous
import functools

import jax
import jax.numpy as jnp
from jax import lax
from jax.experimental import pallas as pl
from jax.experimental.pallas import tpu as pltpu
from jax.experimental.pallas import tpu_sc as plsc

N = 10000
E = 320000
D = 128

NC = 2
NS = 16
NW = NC * NS

EPW = E // NW
K = 80
CHUNKS = EPW // K
GROUPS = 5
CPG = CHUNKS // GROUPS
RPT = 624
TAIL = N - NS * RPT
ZR = 48
CW = 16


def _sc_segment_sum(x, src_r, dst_r):
    mesh = plsc.VectorSubcoreMesh(
        core_axis_name="c", subcore_axis_name="s", num_cores=NC, num_subcores=NS
    )

    @functools.partial(
        pl.kernel,
        out_type=[
            jax.ShapeDtypeStruct((NC, N, D), jnp.float32),
            jax.ShapeDtypeStruct((NC, N, CW), jnp.float32),
        ],
        mesh=mesh,
        scratch_types=[
            pltpu.VMEM_SHARED((N, D), jnp.float32),
            pltpu.VMEM_SHARED((N, CW), jnp.float32),
            pltpu.VMEM((CPG, K), jnp.int32),
            pltpu.VMEM((CPG, K), jnp.int32),
            pltpu.VMEM((K, D), jnp.float32),
            pltpu.VMEM((K, CW), jnp.float32),
            pltpu.VMEM((ZR, D), jnp.float32),
            pltpu.VMEM((ZR, CW), jnp.float32),
            pltpu.SemaphoreType.DMA,
        ],
        compiler_params=pltpu.CompilerParams(use_tc_tiling_on_sc=False),
    )
    def seg_sum(x_hbm, src_hbm, dst_hbm, agg_out, cnt_out,
                agg_sh, cnt_sh, src_v, dst_v, rows_v, ones_v, zbuf, zcnt, sem):
        c = lax.axis_index("c")
        s = lax.axis_index("s")
        wid = c * NS + s

        zero16 = jnp.zeros((16,), jnp.float32)
        one16 = jnp.ones((16,), jnp.float32)

        def zb_body(i, carry):
            for j in range(D // 16):
                zbuf[i, pl.ds(j * 16, 16)] = zero16
            return carry

        lax.fori_loop(0, ZR, zb_body, 0)

        def zc_body(i, carry):
            zcnt[i, :] = zero16
            return carry

        lax.fori_loop(0, ZR, zc_body, 0)

        def on_body(i, carry):
            ones_v[i, :] = one16
            return carry

        lax.fori_loop(0, K, on_body, 0)

        row0 = s * RPT
        for t in range(RPT // ZR):
            pltpu.sync_copy(zbuf, agg_sh.at[pl.ds(row0 + t * ZR, ZR)])
            pltpu.sync_copy(zcnt, cnt_sh.at[pl.ds(row0 + t * ZR, ZR)])

        @pl.when(s == NS - 1)
        def _zero_tail():
            pltpu.sync_copy(zbuf.at[pl.ds(0, TAIL)], agg_sh.at[pl.ds(NS * RPT, TAIL)])
            pltpu.sync_copy(zcnt.at[pl.ds(0, TAIL)], cnt_sh.at[pl.ds(NS * RPT, TAIL)])

        plsc.subcore_barrier()

        def group_body(g, carry):
            pltpu.sync_copy(src_hbm.at[wid, g], src_v)
            pltpu.sync_copy(dst_hbm.at[wid, g], dst_v)

            def chunk_body(j, carry):
                pltpu.async_copy(x_hbm.at[src_v.at[j]], rows_v, sem).wait()
                pltpu.sync_copy(rows_v, agg_sh.at[dst_v.at[j]], add=True)
                pltpu.sync_copy(ones_v, cnt_sh.at[dst_v.at[j]], add=True)
                return carry

            return lax.fori_loop(0, CPG, chunk_body, carry)

        lax.fori_loop(0, GROUPS, group_body, 0)
        plsc.subcore_barrier()

        def wr_body(t, carry):
            r = row0 + t * ZR
            pltpu.sync_copy(agg_sh.at[pl.ds(r, ZR)], zbuf)
            pltpu.sync_copy(zbuf, agg_out.at[c, pl.ds(r, ZR)])
            pltpu.sync_copy(cnt_sh.at[pl.ds(r, ZR)], zcnt)
            pltpu.sync_copy(zcnt, cnt_out.at[c, pl.ds(r, ZR)])
            return carry

        lax.fori_loop(0, RPT // ZR, wr_body, 0)

        @pl.when(s == NS - 1)
        def _write_tail():
            pltpu.sync_copy(agg_sh.at[pl.ds(NS * RPT, TAIL)], zbuf.at[pl.ds(0, TAIL)])
            pltpu.sync_copy(zbuf.at[pl.ds(0, TAIL)], agg_out.at[c, pl.ds(NS * RPT, TAIL)])
            pltpu.sync_copy(cnt_sh.at[pl.ds(NS * RPT, TAIL)], zcnt.at[pl.ds(0, TAIL)])
            pltpu.sync_copy(zcnt.at[pl.ds(0, TAIL)], cnt_out.at[c, pl.ds(NS * RPT, TAIL)])

    return seg_sum(x, src_r, dst_r)


def _tc_body(agg_ref, cnt_ref, x_ref, wl_ref, wr_ref, out_ref):
    agg = agg_ref[0] + agg_ref[1]
    cnt = cnt_ref[0, :, 0:1] + cnt_ref[1, :, 0:1]
    mean = agg / jnp.maximum(cnt, 1.0)
    dn = (((1,), (1,)), ((), ()))
    out_ref[...] = (
        lax.dot_general(mean, wl_ref[...], dn, preferred_element_type=jnp.float32)
        + lax.dot_general(x_ref[...], wr_ref[...], dn, preferred_element_type=jnp.float32)
    )


def _tc_combine(agg2, cnt2, x, W_l, W_r):
    B = 1000
    grid = (N // B,)
    return pl.pallas_call(
        _tc_body,
        grid=grid,
        in_specs=[
            pl.BlockSpec((NC, B, D), lambda i: (0, i, 0)),
            pl.BlockSpec((NC, B, CW), lambda i: (0, i, 0)),
            pl.BlockSpec((B, D), lambda i: (i, 0)),
            pl.BlockSpec((D, D), lambda i: (0, 0)),
            pl.BlockSpec((D, D), lambda i: (0, 0)),
        ],
        out_specs=pl.BlockSpec((B, D), lambda i: (i, 0)),
        out_shape=jax.ShapeDtypeStruct((N, D), jnp.float32),
    )(agg2, cnt2, x, W_l, W_r)


def kernel(x, edge_index, W_l, W_r):
    src_r = edge_index[0].reshape(NW, GROUPS, CPG, K)
    dst_r = edge_index[1].reshape(NW, GROUPS, CPG, K)
    agg2, cnt2 = _sc_segment_sum(x, src_r, dst_r)
    return _tc_combine(agg2, cnt2, x, W_l, W_r)


# speedup vs baseline: 10.6602x; 1.2980x over previous
import functools

import jax
import jax.numpy as jnp
from jax import lax
from jax.experimental import pallas as pl
from jax.experimental.pallas import tpu as pltpu
from jax.experimental.pallas import tpu_sc as plsc

N = 10000
E = 320000
D = 128

NC = 2
NS = 16
NW = NC * NS

EPW = E // NW
K = 100
GROUPS = 5
CPG = 20
PAIRS = CPG // 2
RPT = 624
TAIL = N - NS * RPT
ZR = 24
CW = 16
LAST = PAIRS - 1


def _sc_segment_sum(x, src_r, dst_r):
    mesh = plsc.VectorSubcoreMesh(
        core_axis_name="c", subcore_axis_name="s", num_cores=NC, num_subcores=NS
    )

    @functools.partial(
        pl.kernel,
        out_type=[
            jax.ShapeDtypeStruct((NC, N, D), jnp.float32),
            jax.ShapeDtypeStruct((NC, N, CW), jnp.float32),
        ],
        mesh=mesh,
        scratch_types=[
            pltpu.VMEM_SHARED((N, D), jnp.float32),
            pltpu.VMEM_SHARED((N, CW), jnp.float32),
            pltpu.VMEM((CPG, K), jnp.int32),
            pltpu.VMEM((CPG, K), jnp.int32),
            pltpu.VMEM((K, D), jnp.float32),
            pltpu.VMEM((K, D), jnp.float32),
            pltpu.VMEM((K, CW), jnp.float32),
            pltpu.VMEM((ZR, D), jnp.float32),
            pltpu.VMEM((ZR, CW), jnp.float32),
            pltpu.SemaphoreType.DMA,
            pltpu.SemaphoreType.DMA,
            pltpu.SemaphoreType.DMA,
            pltpu.SemaphoreType.DMA,
            pltpu.SemaphoreType.DMA,
            pltpu.SemaphoreType.DMA,
        ],
        compiler_params=pltpu.CompilerParams(use_tc_tiling_on_sc=False),
    )
    def seg_sum(x_hbm, src_hbm, dst_hbm, agg_out, cnt_out,
                agg_sh, cnt_sh, src_v, dst_v, rows0, rows1, ones_v, zbuf, zcnt,
                g0, g1, sa0, sa1, sc0, sc1):
        c = lax.axis_index("c")
        s = lax.axis_index("s")
        wid = c * NS + s

        zero16 = jnp.zeros((16,), jnp.float32)
        one16 = jnp.ones((16,), jnp.float32)

        def zb_body(i, carry):
            for j in range(D // 16):
                zbuf[i, pl.ds(j * 16, 16)] = zero16
            return carry

        lax.fori_loop(0, ZR, zb_body, 0)

        def zc_body(i, carry):
            zcnt[i, :] = zero16
            return carry

        lax.fori_loop(0, ZR, zc_body, 0)

        def on_body(i, carry):
            ones_v[i, :] = one16
            return carry

        lax.fori_loop(0, K, on_body, 0)

        row0 = s * RPT

        def zs_body(t, carry):
            pltpu.sync_copy(zbuf, agg_sh.at[pl.ds(row0 + t * ZR, ZR)])
            pltpu.sync_copy(zcnt, cnt_sh.at[pl.ds(row0 + t * ZR, ZR)])
            return carry

        lax.fori_loop(0, RPT // ZR, zs_body, 0)

        @pl.when(s == NS - 1)
        def _zero_tail():
            pltpu.sync_copy(zbuf.at[pl.ds(0, TAIL)], agg_sh.at[pl.ds(NS * RPT, TAIL)])
            pltpu.sync_copy(zcnt.at[pl.ds(0, TAIL)], cnt_sh.at[pl.ds(NS * RPT, TAIL)])

        plsc.subcore_barrier()

        def wait_gather(rows, sem):
            pltpu.make_async_copy(x_hbm.at[src_v.at[0]], rows, sem).wait()

        def issue_scatter(rows, j, sema, semc):
            pltpu.async_copy(rows, agg_sh.at[dst_v.at[j]], sema, add=True)
            pltpu.async_copy(ones_v, cnt_sh.at[dst_v.at[j]], semc, add=True)

        def wait_scatter(rows, j, sema, semc):
            pltpu.make_async_copy(rows, agg_sh.at[dst_v.at[j]], sema).wait()
            pltpu.make_async_copy(ones_v, cnt_sh.at[dst_v.at[j]], semc).wait()

        def group_body(g, carry):
            pltpu.sync_copy(src_hbm.at[wid, g], src_v)
            pltpu.sync_copy(dst_hbm.at[wid, g], dst_v)
            pltpu.async_copy(x_hbm.at[src_v.at[0]], rows0, g0)

            def pair_body(t, carry):
                a = 2 * t
                b = a + 1
                wait_gather(rows0, g0)
                issue_scatter(rows0, a, sa0, sc0)

                @pl.when(t > 0)
                def _drain1():
                    wait_scatter(rows1, b, sa1, sc1)

                pltpu.async_copy(x_hbm.at[src_v.at[b]], rows1, g1)
                wait_gather(rows1, g1)
                issue_scatter(rows1, b, sa1, sc1)

                @pl.when(t < LAST)
                def _next0():
                    wait_scatter(rows0, a, sa0, sc0)
                    pltpu.async_copy(x_hbm.at[src_v.at[a + 2]], rows0, g0)

                return carry

            lax.fori_loop(0, PAIRS, pair_body, carry)
            wait_scatter(rows0, 0, sa0, sc0)
            wait_scatter(rows1, 1, sa1, sc1)
            return carry

        lax.fori_loop(0, GROUPS, group_body, 0)
        plsc.subcore_barrier()

        def wr_body(t, carry):
            r = row0 + t * ZR
            pltpu.sync_copy(agg_sh.at[pl.ds(r, ZR)], zbuf)
            pltpu.sync_copy(zbuf, agg_out.at[c, pl.ds(r, ZR)])
            pltpu.sync_copy(cnt_sh.at[pl.ds(r, ZR)], zcnt)
            pltpu.sync_copy(zcnt, cnt_out.at[c, pl.ds(r, ZR)])
            return carry

        lax.fori_loop(0, RPT // ZR, wr_body, 0)

        @pl.when(s == NS - 1)
        def _write_tail():
            pltpu.sync_copy(agg_sh.at[pl.ds(NS * RPT, TAIL)], zbuf.at[pl.ds(0, TAIL)])
            pltpu.sync_copy(zbuf.at[pl.ds(0, TAIL)], agg_out.at[c, pl.ds(NS * RPT, TAIL)])
            pltpu.sync_copy(cnt_sh.at[pl.ds(NS * RPT, TAIL)], zcnt.at[pl.ds(0, TAIL)])
            pltpu.sync_copy(zcnt.at[pl.ds(0, TAIL)], cnt_out.at[c, pl.ds(NS * RPT, TAIL)])

    return seg_sum(x, src_r, dst_r)


def _tc_body(agg_ref, cnt_ref, x_ref, wl_ref, wr_ref, out_ref):
    agg = agg_ref[0] + agg_ref[1]
    cnt = cnt_ref[0, :, 0:1] + cnt_ref[1, :, 0:1]
    mean = agg / jnp.maximum(cnt, 1.0)
    dn = (((1,), (1,)), ((), ()))
    out_ref[...] = (
        lax.dot_general(mean, wl_ref[...], dn, preferred_element_type=jnp.float32)
        + lax.dot_general(x_ref[...], wr_ref[...], dn, preferred_element_type=jnp.float32)
    )


def _tc_combine(agg2, cnt2, x, W_l, W_r):
    B = 1000
    grid = (N // B,)
    return pl.pallas_call(
        _tc_body,
        grid=grid,
        in_specs=[
            pl.BlockSpec((NC, B, D), lambda i: (0, i, 0)),
            pl.BlockSpec((NC, B, CW), lambda i: (0, i, 0)),
            pl.BlockSpec((B, D), lambda i: (i, 0)),
            pl.BlockSpec((D, D), lambda i: (0, 0)),
            pl.BlockSpec((D, D), lambda i: (0, 0)),
        ],
        out_specs=pl.BlockSpec((B, D), lambda i: (i, 0)),
        out_shape=jax.ShapeDtypeStruct((N, D), jnp.float32),
    )(agg2, cnt2, x, W_l, W_r)


def kernel(x, edge_index, W_l, W_r):
    src_r = edge_index[0].reshape(NW, GROUPS, CPG, K)
    dst_r = edge_index[1].reshape(NW, GROUPS, CPG, K)
    agg2, cnt2 = _sc_segment_sum(x, src_r, dst_r)
    return _tc_combine(agg2, cnt2, x, W_l, W_r)


# speedup vs baseline: 11.2200x; 1.0525x over previous
import functools

import jax
import jax.numpy as jnp
from jax import lax
from jax.experimental import pallas as pl
from jax.experimental.pallas import tpu as pltpu
from jax.experimental.pallas import tpu_sc as plsc

N = 10000
E = 320000
D = 128

NC = 2
NS = 16
NW = NC * NS

EPW = E // NW
K = 100
GROUPS = 5
CPG = 20
EPG = CPG * K
PAIRS = CPG // 2
RPT = 624
TAIL = N - NS * RPT
ZR = 24
LAST = PAIRS - 1


def _sc_segment_sum(x, src_r, dst_r, dst_f_r):
    mesh = plsc.VectorSubcoreMesh(
        core_axis_name="c", subcore_axis_name="s", num_cores=NC, num_subcores=NS
    )

    @functools.partial(
        pl.kernel,
        out_type=[
            jax.ShapeDtypeStruct((NC, N, D), jnp.float32),
            jax.ShapeDtypeStruct((NC, N), jnp.float32),
        ],
        mesh=mesh,
        scratch_types=[
            pltpu.VMEM_SHARED((N, D), jnp.float32),
            pltpu.VMEM_SHARED((N,), jnp.float32),
            pltpu.VMEM((CPG, K), jnp.int32),
            pltpu.VMEM((CPG, K), jnp.int32),
            pltpu.VMEM((EPG,), jnp.int32),
            pltpu.VMEM((K, D), jnp.float32),
            pltpu.VMEM((K, D), jnp.float32),
            pltpu.VMEM((EPG,), jnp.float32),
            pltpu.VMEM((ZR, D), jnp.float32),
            pltpu.VMEM((RPT,), jnp.float32),
            pltpu.SemaphoreType.DMA,
            pltpu.SemaphoreType.DMA,
            pltpu.SemaphoreType.DMA,
            pltpu.SemaphoreType.DMA,
            pltpu.SemaphoreType.DMA,
        ],
        compiler_params=pltpu.CompilerParams(use_tc_tiling_on_sc=False),
    )
    def seg_sum(x_hbm, src_hbm, dst_hbm, dstf_hbm, agg_out, cnt_out,
                agg_sh, cnt_sh, src_v, dst_v, dst_f, rows0, rows1, ones_v,
                zbuf, zcnt, g0, g1, sa0, sa1, scnt):
        c = lax.axis_index("c")
        s = lax.axis_index("s")
        wid = c * NS + s

        zero16 = jnp.zeros((16,), jnp.float32)
        one16 = jnp.ones((16,), jnp.float32)

        def zb_body(i, carry):
            for j in range(D // 16):
                zbuf[i, pl.ds(j * 16, 16)] = zero16
            return carry

        lax.fori_loop(0, ZR, zb_body, 0)

        def zc_body(i, carry):
            zcnt[pl.ds(i * 16, 16)] = zero16
            return carry

        lax.fori_loop(0, RPT // 16, zc_body, 0)

        def on_body(i, carry):
            ones_v[pl.ds(i * 16, 16)] = one16
            return carry

        lax.fori_loop(0, EPG // 16, on_body, 0)

        row0 = s * RPT

        def zs_body(t, carry):
            pltpu.sync_copy(zbuf, agg_sh.at[pl.ds(row0 + t * ZR, ZR)])
            return carry

        lax.fori_loop(0, RPT // ZR, zs_body, 0)
        pltpu.sync_copy(zcnt, cnt_sh.at[pl.ds(row0, RPT)])

        @pl.when(s == NS - 1)
        def _zero_tail():
            pltpu.sync_copy(zbuf.at[pl.ds(0, TAIL)], agg_sh.at[pl.ds(NS * RPT, TAIL)])
            pltpu.sync_copy(zcnt.at[pl.ds(0, TAIL)], cnt_sh.at[pl.ds(NS * RPT, TAIL)])

        plsc.subcore_barrier()

        def wait_gather(rows, sem):
            pltpu.make_async_copy(x_hbm.at[src_v.at[0]], rows, sem).wait()

        def issue_scatter(rows, j, sema):
            pltpu.async_copy(rows, agg_sh.at[dst_v.at[j]], sema, add=True)

        def wait_scatter(rows, sema):
            pltpu.make_async_copy(rows, agg_sh.at[dst_v.at[0]], sema).wait()

        def group_body(g, carry):
            pltpu.sync_copy(src_hbm.at[wid, g], src_v)
            pltpu.sync_copy(dst_hbm.at[wid, g], dst_v)
            pltpu.sync_copy(dstf_hbm.at[wid, g], dst_f)
            pltpu.async_copy(ones_v, cnt_sh.at[dst_f], scnt, add=True)
            pltpu.async_copy(x_hbm.at[src_v.at[0]], rows0, g0)

            def pair_body(t, carry):
                a = 2 * t
                b = a + 1
                wait_gather(rows0, g0)
                issue_scatter(rows0, a, sa0)

                @pl.when(t > 0)
                def _drain1():
                    wait_scatter(rows1, sa1)

                pltpu.async_copy(x_hbm.at[src_v.at[b]], rows1, g1)
                wait_gather(rows1, g1)
                issue_scatter(rows1, b, sa1)

                @pl.when(t < LAST)
                def _next0():
                    wait_scatter(rows0, sa0)
                    pltpu.async_copy(x_hbm.at[src_v.at[a + 2]], rows0, g0)

                return carry

            lax.fori_loop(0, PAIRS, pair_body, carry)
            wait_scatter(rows0, sa0)
            wait_scatter(rows1, sa1)
            pltpu.make_async_copy(ones_v, cnt_sh.at[dst_f], scnt).wait()
            return carry

        lax.fori_loop(0, GROUPS, group_body, 0)
        plsc.subcore_barrier()

        def wr_body(t, carry):
            r = row0 + t * ZR
            pltpu.sync_copy(agg_sh.at[pl.ds(r, ZR)], zbuf)
            pltpu.sync_copy(zbuf, agg_out.at[c, pl.ds(r, ZR)])
            return carry

        lax.fori_loop(0, RPT // ZR, wr_body, 0)
        pltpu.sync_copy(cnt_sh.at[pl.ds(row0, RPT)], zcnt)
        pltpu.sync_copy(zcnt, cnt_out.at[c, pl.ds(row0, RPT)])

        @pl.when(s == NS - 1)
        def _write_tail():
            pltpu.sync_copy(agg_sh.at[pl.ds(NS * RPT, TAIL)], zbuf.at[pl.ds(0, TAIL)])
            pltpu.sync_copy(zbuf.at[pl.ds(0, TAIL)], agg_out.at[c, pl.ds(NS * RPT, TAIL)])
            pltpu.sync_copy(cnt_sh.at[pl.ds(NS * RPT, TAIL)], zcnt.at[pl.ds(0, TAIL)])
            pltpu.sync_copy(zcnt.at[pl.ds(0, TAIL)], cnt_out.at[c, pl.ds(NS * RPT, TAIL)])

    return seg_sum(x, src_r, dst_r, dst_f_r)


def _tc_body(agg_ref, cnt_ref, x_ref, wl_ref, wr_ref, out_ref):
    agg = agg_ref[0] + agg_ref[1]
    cnt = cnt_ref[0, 0, :] + cnt_ref[0, 1, :]
    mean = agg / jnp.maximum(cnt, 1.0)[:, None]
    dn = (((1,), (1,)), ((), ()))
    out_ref[...] = (
        lax.dot_general(mean, wl_ref[...], dn, preferred_element_type=jnp.float32)
        + lax.dot_general(x_ref[...], wr_ref[...], dn, preferred_element_type=jnp.float32)
    )


def _tc_combine(agg2, cnt2, x, W_l, W_r):
    B = 1000
    grid = (N // B,)
    return pl.pallas_call(
        _tc_body,
        grid=grid,
        in_specs=[
            pl.BlockSpec((NC, B, D), lambda i: (0, i, 0)),
            pl.BlockSpec((1, NC, B), lambda i: (i, 0, 0)),
            pl.BlockSpec((B, D), lambda i: (i, 0)),
            pl.BlockSpec((D, D), lambda i: (0, 0)),
            pl.BlockSpec((D, D), lambda i: (0, 0)),
        ],
        out_specs=pl.BlockSpec((B, D), lambda i: (i, 0)),
        out_shape=jax.ShapeDtypeStruct((N, D), jnp.float32),
    )(agg2, cnt2.reshape(NC, N // B, B).transpose(1, 0, 2), x, W_l, W_r)


def kernel(x, edge_index, W_l, W_r):
    src_r = edge_index[0].reshape(NW, GROUPS, CPG, K)
    dst_r = edge_index[1].reshape(NW, GROUPS, CPG, K)
    dst_f_r = edge_index[1].reshape(NW, GROUPS, EPG)
    agg2, cnt2 = _sc_segment_sum(x, src_r, dst_r, dst_f_r)
    return _tc_combine(agg2, cnt2, x, W_l, W_r)


# speedup vs baseline: 14.1929x; 1.2650x over previous
import functools

import jax
import jax.numpy as jnp
from jax import lax
from jax.experimental import pallas as pl
from jax.experimental.pallas import tpu as pltpu
from jax.experimental.pallas import tpu_sc as plsc

N = 10000
E = 320000
D = 128

NC = 2
NS = 16
NW = NC * NS

EPW = E // NW
K = 100
GROUPS = 5
CPG = 20
EPG = CPG * K
SLOTS = 3
RPT = 624
TAIL = N - NS * RPT
ZR = 24


def _sc_segment_sum(x, src_r, dst_r):
    mesh = plsc.VectorSubcoreMesh(
        core_axis_name="c", subcore_axis_name="s", num_cores=NC, num_subcores=NS
    )

    @functools.partial(
        pl.kernel,
        out_type=[
            jax.ShapeDtypeStruct((NC, N, D), jnp.float32),
            jax.ShapeDtypeStruct((NC, N), jnp.float32),
        ],
        mesh=mesh,
        scratch_types=[
            pltpu.VMEM_SHARED((N, D), jnp.float32),
            pltpu.VMEM_SHARED((N,), jnp.float32),
            pltpu.VMEM((CPG, K), jnp.int32),
            pltpu.VMEM((CPG, K), jnp.int32),
            pltpu.VMEM((SLOTS, K, D), jnp.float32),
            pltpu.VMEM((K,), jnp.float32),
            pltpu.VMEM((ZR, D), jnp.float32),
            pltpu.VMEM((RPT,), jnp.float32),
            pltpu.SemaphoreType.DMA((SLOTS,)),
            pltpu.SemaphoreType.DMA((SLOTS,)),
            pltpu.SemaphoreType.DMA((SLOTS,)),
        ],
        compiler_params=pltpu.CompilerParams(use_tc_tiling_on_sc=False),
    )
    def seg_sum(x_hbm, src_hbm, dst_hbm, agg_out, cnt_out,
                agg_sh, cnt_sh, src_v, dst_v, rows, ones_v,
                zbuf, zcnt, gsem, asem, csem):
        c = lax.axis_index("c")
        s = lax.axis_index("s")
        wid = c * NS + s

        zero16 = jnp.zeros((16,), jnp.float32)
        one16 = jnp.ones((16,), jnp.float32)

        def zb_body(i, carry):
            for j in range(D // 16):
                zbuf[i, pl.ds(j * 16, 16)] = zero16
            return carry

        lax.fori_loop(0, ZR, zb_body, 0)

        def zc_body(i, carry):
            zcnt[pl.ds(i * 16, 16)] = zero16
            return carry

        lax.fori_loop(0, RPT // 16, zc_body, 0)

        def on_body(i, carry):
            ones_v[pl.ds(i * 16, 16)] = one16
            return carry

        lax.fori_loop(0, K // 16, on_body, 0)
        ones_v[pl.ds(K - 16, 16)] = one16

        row0 = s * RPT

        def zs_body(t, carry):
            pltpu.sync_copy(zbuf, agg_sh.at[pl.ds(row0 + t * ZR, ZR)])
            return carry

        lax.fori_loop(0, RPT // ZR, zs_body, 0)
        pltpu.sync_copy(zcnt, cnt_sh.at[pl.ds(row0, RPT)])

        @pl.when(s == NS - 1)
        def _zero_tail():
            pltpu.sync_copy(zbuf.at[pl.ds(0, TAIL)], agg_sh.at[pl.ds(NS * RPT, TAIL)])
            pltpu.sync_copy(zcnt.at[pl.ds(0, TAIL)], cnt_sh.at[pl.ds(NS * RPT, TAIL)])

        plsc.subcore_barrier()

        def issue_gather(j, slot):
            pltpu.async_copy(x_hbm.at[src_v.at[j]], rows.at[slot], gsem.at[slot])

        def wait_gather(slot):
            pltpu.make_async_copy(
                x_hbm.at[src_v.at[0]], rows.at[slot], gsem.at[slot]
            ).wait()

        def issue_scatter(j, slot):
            pltpu.async_copy(rows.at[slot], agg_sh.at[dst_v.at[j]], asem.at[slot],
                             add=True)
            pltpu.async_copy(ones_v, cnt_sh.at[dst_v.at[j]], csem.at[slot],
                             add=True)

        def wait_scatter(slot):
            pltpu.make_async_copy(rows.at[slot], agg_sh.at[dst_v.at[0]],
                                  asem.at[slot]).wait()
            pltpu.make_async_copy(ones_v, cnt_sh.at[dst_v.at[0]],
                                  csem.at[slot]).wait()

        def group_body(g, carry):
            pltpu.sync_copy(src_hbm.at[wid, g], src_v)
            pltpu.sync_copy(dst_hbm.at[wid, g], dst_v)
            issue_gather(0, 0)
            issue_gather(1, 1)

            def chunk_body(t, carry):
                slot = lax.rem(t, SLOTS)
                wait_gather(slot)
                issue_scatter(t, slot)

                @pl.when(t >= 1)
                def _drain_prev():
                    wait_scatter(lax.rem(t + SLOTS - 1, SLOTS))

                @pl.when(t + 2 < CPG)
                def _prefetch():
                    issue_gather(t + 2, lax.rem(t + 2, SLOTS))

                return carry

            lax.fori_loop(0, CPG, chunk_body, carry)
            wait_scatter(lax.rem(CPG - 1, SLOTS))
            return carry

        lax.fori_loop(0, GROUPS, group_body, 0)
        plsc.subcore_barrier()

        def wr_body(t, carry):
            r = row0 + t * ZR
            pltpu.sync_copy(agg_sh.at[pl.ds(r, ZR)], zbuf)
            pltpu.sync_copy(zbuf, agg_out.at[c, pl.ds(r, ZR)])
            return carry

        lax.fori_loop(0, RPT // ZR, wr_body, 0)
        pltpu.sync_copy(cnt_sh.at[pl.ds(row0, RPT)], zcnt)
        pltpu.sync_copy(zcnt, cnt_out.at[c, pl.ds(row0, RPT)])

        @pl.when(s == NS - 1)
        def _write_tail():
            pltpu.sync_copy(agg_sh.at[pl.ds(NS * RPT, TAIL)], zbuf.at[pl.ds(0, TAIL)])
            pltpu.sync_copy(zbuf.at[pl.ds(0, TAIL)], agg_out.at[c, pl.ds(NS * RPT, TAIL)])
            pltpu.sync_copy(cnt_sh.at[pl.ds(NS * RPT, TAIL)], zcnt.at[pl.ds(0, TAIL)])
            pltpu.sync_copy(zcnt.at[pl.ds(0, TAIL)], cnt_out.at[c, pl.ds(NS * RPT, TAIL)])

    return seg_sum(x, src_r, dst_r)


def _tc_body(agg_ref, cnt_ref, x_ref, wl_ref, wr_ref, out_ref):
    agg = agg_ref[0] + agg_ref[1]
    cnt = cnt_ref[0, 0, :] + cnt_ref[0, 1, :]
    mean = agg / jnp.maximum(cnt, 1.0)[:, None]
    dn = (((1,), (1,)), ((), ()))
    out_ref[...] = (
        lax.dot_general(mean, wl_ref[...], dn, preferred_element_type=jnp.float32)
        + lax.dot_general(x_ref[...], wr_ref[...], dn, preferred_element_type=jnp.float32)
    )


def _tc_combine(agg2, cnt2, x, W_l, W_r):
    B = 1000
    grid = (N // B,)
    return pl.pallas_call(
        _tc_body,
        grid=grid,
        in_specs=[
            pl.BlockSpec((NC, B, D), lambda i: (0, i, 0)),
            pl.BlockSpec((1, NC, B), lambda i: (i, 0, 0)),
            pl.BlockSpec((B, D), lambda i: (i, 0)),
            pl.BlockSpec((D, D), lambda i: (0, 0)),
            pl.BlockSpec((D, D), lambda i: (0, 0)),
        ],
        out_specs=pl.BlockSpec((B, D), lambda i: (i, 0)),
        out_shape=jax.ShapeDtypeStruct((N, D), jnp.float32),
    )(agg2, cnt2.reshape(NC, N // B, B).transpose(1, 0, 2), x, W_l, W_r)


def kernel(x, edge_index, W_l, W_r):
    src_r = edge_index[0].reshape(NW, GROUPS, CPG, K)
    dst_r = edge_index[1].reshape(NW, GROUPS, CPG, K)
    agg2, cnt2 = _sc_segment_sum(x, src_r, dst_r)
    return _tc_combine(agg2, cnt2, x, W_l, W_r)


# speedup vs baseline: 14.5857x; 1.0277x over previous
import functools

import jax
import jax.numpy as jnp
from jax import lax
from jax.experimental import pallas as pl
from jax.experimental.pallas import tpu as pltpu
from jax.experimental.pallas import tpu_sc as plsc

N = 10000
E = 320000
D = 128

NC = 2
NS = 16
NW = NC * NS

EPW = E // NW
K = 100
GROUPS = 5
CPG = 20
EPG = CPG * K
SLOTS = 3
RPT = 624
TAIL = N - NS * RPT
ZR = 24


def _sc_segment_sum(x, src_r, dst_r):
    mesh = plsc.VectorSubcoreMesh(
        core_axis_name="c", subcore_axis_name="s", num_cores=NC, num_subcores=NS
    )

    @functools.partial(
        pl.kernel,
        out_type=[
            jax.ShapeDtypeStruct((NC, N, D), jnp.float32),
            jax.ShapeDtypeStruct((NC, N), jnp.float32),
        ],
        mesh=mesh,
        scratch_types=[
            pltpu.VMEM_SHARED((N, D), jnp.float32),
            pltpu.VMEM_SHARED((N,), jnp.float32),
            pltpu.VMEM((CPG, K), jnp.int32),
            pltpu.VMEM((CPG, K), jnp.int32),
            pltpu.VMEM((SLOTS, K, D), jnp.float32),
            pltpu.VMEM((K,), jnp.float32),
            pltpu.VMEM((ZR, D), jnp.float32),
            pltpu.VMEM((RPT,), jnp.float32),
            pltpu.SemaphoreType.DMA((SLOTS,)),
            pltpu.SemaphoreType.DMA((SLOTS,)),
            pltpu.SemaphoreType.DMA((SLOTS,)),
            pltpu.SemaphoreType.DMA,
        ],
        compiler_params=pltpu.CompilerParams(use_tc_tiling_on_sc=False),
    )
    def seg_sum(x_hbm, src_hbm, dst_hbm, agg_out, cnt_out,
                agg_sh, cnt_sh, src_v, dst_v, rows, ones_v,
                zbuf, zcnt, gsem, asem, csem, bsem):
        c = lax.axis_index("c")
        s = lax.axis_index("s")
        wid = c * NS + s

        zero16 = jnp.zeros((16,), jnp.float32)
        one16 = jnp.ones((16,), jnp.float32)

        def zb_body(i, carry):
            for j in range(D // 16):
                zbuf[i, pl.ds(j * 16, 16)] = zero16
            return carry

        lax.fori_loop(0, ZR, zb_body, 0)

        def zc_body(i, carry):
            zcnt[pl.ds(i * 16, 16)] = zero16
            return carry

        lax.fori_loop(0, RPT // 16, zc_body, 0)

        def on_body(i, carry):
            ones_v[pl.ds(i * 16, 16)] = one16
            return carry

        lax.fori_loop(0, K // 16, on_body, 0)
        ones_v[pl.ds(K - 16, 16)] = one16

        row0 = s * RPT

        def zs_body(t, carry):
            pltpu.async_copy(zbuf, agg_sh.at[pl.ds(row0 + t * ZR, ZR)], bsem)
            return carry

        lax.fori_loop(0, RPT // ZR, zs_body, 0)
        pltpu.async_copy(zcnt, cnt_sh.at[pl.ds(row0, RPT)], bsem)

        @pl.when(s == NS - 1)
        def _zero_tail():
            pltpu.sync_copy(zbuf.at[pl.ds(0, TAIL)], agg_sh.at[pl.ds(NS * RPT, TAIL)])
            pltpu.sync_copy(zcnt.at[pl.ds(0, TAIL)], cnt_sh.at[pl.ds(NS * RPT, TAIL)])

        def zs_drain(t, carry):
            pltpu.make_async_copy(zbuf, agg_sh.at[pl.ds(row0, ZR)], bsem).wait()
            return carry

        lax.fori_loop(0, RPT // ZR, zs_drain, 0)
        pltpu.make_async_copy(zcnt, cnt_sh.at[pl.ds(row0, RPT)], bsem).wait()
        plsc.subcore_barrier()

        def issue_gather(j, slot):
            pltpu.async_copy(x_hbm.at[src_v.at[j]], rows.at[slot], gsem.at[slot])

        def wait_gather(slot):
            pltpu.make_async_copy(
                x_hbm.at[src_v.at[0]], rows.at[slot], gsem.at[slot]
            ).wait()

        def issue_scatter(j, slot):
            pltpu.async_copy(rows.at[slot], agg_sh.at[dst_v.at[j]], asem.at[slot],
                             add=True)
            pltpu.async_copy(ones_v, cnt_sh.at[dst_v.at[j]], csem.at[slot],
                             add=True)

        def wait_scatter(slot):
            pltpu.make_async_copy(rows.at[slot], agg_sh.at[dst_v.at[0]],
                                  asem.at[slot]).wait()
            pltpu.make_async_copy(ones_v, cnt_sh.at[dst_v.at[0]],
                                  csem.at[slot]).wait()

        def group_body(g, carry):
            pltpu.sync_copy(src_hbm.at[wid, g], src_v)
            pltpu.sync_copy(dst_hbm.at[wid, g], dst_v)
            issue_gather(0, 0)
            issue_gather(1, 1)

            def chunk_body(t, carry):
                slot = lax.rem(t, SLOTS)
                wait_gather(slot)
                issue_scatter(t, slot)

                @pl.when(t >= 1)
                def _drain_prev():
                    wait_scatter(lax.rem(t + SLOTS - 1, SLOTS))

                @pl.when(t + 2 < CPG)
                def _prefetch():
                    issue_gather(t + 2, lax.rem(t + 2, SLOTS))

                return carry

            lax.fori_loop(0, CPG, chunk_body, carry)
            wait_scatter(lax.rem(CPG - 1, SLOTS))
            return carry

        lax.fori_loop(0, GROUPS, group_body, 0)
        plsc.subcore_barrier()

        pltpu.async_copy(
            agg_sh.at[pl.ds(row0, RPT)], agg_out.at[c, pl.ds(row0, RPT)], bsem
        )
        pltpu.async_copy(
            cnt_sh.at[pl.ds(row0, RPT)], cnt_out.at[c, pl.ds(row0, RPT)], bsem
        )

        @pl.when(s == NS - 1)
        def _write_tail():
            pltpu.sync_copy(
                agg_sh.at[pl.ds(NS * RPT, TAIL)], agg_out.at[c, pl.ds(NS * RPT, TAIL)]
            )
            pltpu.sync_copy(
                cnt_sh.at[pl.ds(NS * RPT, TAIL)], cnt_out.at[c, pl.ds(NS * RPT, TAIL)]
            )

        pltpu.make_async_copy(
            agg_sh.at[pl.ds(row0, RPT)], agg_out.at[c, pl.ds(row0, RPT)], bsem
        ).wait()
        pltpu.make_async_copy(
            cnt_sh.at[pl.ds(row0, RPT)], cnt_out.at[c, pl.ds(row0, RPT)], bsem
        ).wait()

    return seg_sum(x, src_r, dst_r)


def _tc_body(agg_ref, cnt_ref, x_ref, wl_ref, wr_ref, out_ref):
    agg = agg_ref[0] + agg_ref[1]
    cnt = cnt_ref[0, 0, :] + cnt_ref[0, 1, :]
    mean = agg / jnp.maximum(cnt, 1.0)[:, None]
    dn = (((1,), (1,)), ((), ()))
    out_ref[...] = (
        lax.dot_general(mean, wl_ref[...], dn, preferred_element_type=jnp.float32)
        + lax.dot_general(x_ref[...], wr_ref[...], dn, preferred_element_type=jnp.float32)
    )


def _tc_combine(agg2, cnt2, x, W_l, W_r):
    B = 1000
    grid = (N // B,)
    return pl.pallas_call(
        _tc_body,
        grid=grid,
        in_specs=[
            pl.BlockSpec((NC, B, D), lambda i: (0, i, 0)),
            pl.BlockSpec((1, NC, B), lambda i: (i, 0, 0)),
            pl.BlockSpec((B, D), lambda i: (i, 0)),
            pl.BlockSpec((D, D), lambda i: (0, 0)),
            pl.BlockSpec((D, D), lambda i: (0, 0)),
        ],
        out_specs=pl.BlockSpec((B, D), lambda i: (i, 0)),
        out_shape=jax.ShapeDtypeStruct((N, D), jnp.float32),
    )(agg2, cnt2.reshape(NC, N // B, B).transpose(1, 0, 2), x, W_l, W_r)


def kernel(x, edge_index, W_l, W_r):
    src_r = edge_index[0].reshape(NW, GROUPS, CPG, K)
    dst_r = edge_index[1].reshape(NW, GROUPS, CPG, K)
    agg2, cnt2 = _sc_segment_sum(x, src_r, dst_r)
    return _tc_combine(agg2, cnt2, x, W_l, W_r)


# speedup vs baseline: 15.1445x; 1.0383x over previous
import functools

import jax
import jax.numpy as jnp
from jax import lax
from jax.experimental import pallas as pl
from jax.experimental.pallas import tpu as pltpu
from jax.experimental.pallas import tpu_sc as plsc

N = 10000
E = 320000
D = 128

NC = 2
NS = 16
NW = NC * NS

EPW = E // NW
K = 80
GROUPS = 5
CPG = 25
EPG = CPG * K
SLOTS = 4
RPT = 624
TAIL = N - NS * RPT
ZR = 16


def _sc_segment_sum(x, src_r, dst_r):
    mesh = plsc.VectorSubcoreMesh(
        core_axis_name="c", subcore_axis_name="s", num_cores=NC, num_subcores=NS
    )

    @functools.partial(
        pl.kernel,
        out_type=[
            jax.ShapeDtypeStruct((NC, N, D), jnp.float32),
            jax.ShapeDtypeStruct((NC, N), jnp.float32),
        ],
        mesh=mesh,
        scratch_types=[
            pltpu.VMEM_SHARED((N, D), jnp.float32),
            pltpu.VMEM_SHARED((N,), jnp.float32),
            pltpu.VMEM((CPG, K), jnp.int32),
            pltpu.VMEM((CPG, K), jnp.int32),
            pltpu.VMEM((SLOTS, K, D), jnp.float32),
            pltpu.VMEM((K,), jnp.float32),
            pltpu.VMEM((ZR, D), jnp.float32),
            pltpu.VMEM((RPT,), jnp.float32),
            pltpu.SemaphoreType.DMA((SLOTS,)),
            pltpu.SemaphoreType.DMA((SLOTS,)),
            pltpu.SemaphoreType.DMA((SLOTS,)),
            pltpu.SemaphoreType.DMA,
        ],
        compiler_params=pltpu.CompilerParams(use_tc_tiling_on_sc=False),
    )
    def seg_sum(x_hbm, src_hbm, dst_hbm, agg_out, cnt_out,
                agg_sh, cnt_sh, src_v, dst_v, rows, ones_v,
                zbuf, zcnt, gsem, asem, csem, bsem):
        c = lax.axis_index("c")
        s = lax.axis_index("s")
        wid = c * NS + s

        zero16 = jnp.zeros((16,), jnp.float32)
        one16 = jnp.ones((16,), jnp.float32)

        def zb_body(i, carry):
            for j in range(D // 16):
                zbuf[i, pl.ds(j * 16, 16)] = zero16
            return carry

        lax.fori_loop(0, ZR, zb_body, 0)

        def zc_body(i, carry):
            zcnt[pl.ds(i * 16, 16)] = zero16
            return carry

        lax.fori_loop(0, RPT // 16, zc_body, 0)

        def on_body(i, carry):
            ones_v[pl.ds(i * 16, 16)] = one16
            return carry

        lax.fori_loop(0, K // 16, on_body, 0)

        row0 = s * RPT

        def zs_body(t, carry):
            pltpu.async_copy(zbuf, agg_sh.at[pl.ds(row0 + t * ZR, ZR)], bsem)
            return carry

        lax.fori_loop(0, RPT // ZR, zs_body, 0)
        pltpu.async_copy(zcnt, cnt_sh.at[pl.ds(row0, RPT)], bsem)

        @pl.when(s == NS - 1)
        def _zero_tail():
            pltpu.sync_copy(zbuf.at[pl.ds(0, TAIL)], agg_sh.at[pl.ds(NS * RPT, TAIL)])
            pltpu.sync_copy(zcnt.at[pl.ds(0, TAIL)], cnt_sh.at[pl.ds(NS * RPT, TAIL)])

        def zs_drain(t, carry):
            pltpu.make_async_copy(zbuf, agg_sh.at[pl.ds(row0, ZR)], bsem).wait()
            return carry

        lax.fori_loop(0, RPT // ZR, zs_drain, 0)
        pltpu.make_async_copy(zcnt, cnt_sh.at[pl.ds(row0, RPT)], bsem).wait()
        plsc.subcore_barrier()

        def issue_gather(j, slot):
            pltpu.async_copy(x_hbm.at[src_v.at[j]], rows.at[slot], gsem.at[slot])

        def wait_gather(slot):
            pltpu.make_async_copy(
                x_hbm.at[src_v.at[0]], rows.at[slot], gsem.at[slot]
            ).wait()

        def issue_scatter(j, slot):
            pltpu.async_copy(rows.at[slot], agg_sh.at[dst_v.at[j]], asem.at[slot],
                             add=True)
            pltpu.async_copy(ones_v, cnt_sh.at[dst_v.at[j]], csem.at[slot],
                             add=True)

        def wait_scatter(slot):
            pltpu.make_async_copy(rows.at[slot], agg_sh.at[dst_v.at[0]],
                                  asem.at[slot]).wait()
            pltpu.make_async_copy(ones_v, cnt_sh.at[dst_v.at[0]],
                                  csem.at[slot]).wait()

        def group_body(g, carry):
            pltpu.sync_copy(src_hbm.at[wid, g], src_v)
            pltpu.sync_copy(dst_hbm.at[wid, g], dst_v)
            issue_gather(0, 0)
            issue_gather(1, 1)
            issue_gather(2, 2)

            def chunk_body(t, carry):
                slot = lax.rem(t, SLOTS)
                wait_gather(slot)
                issue_scatter(t, slot)

                @pl.when(t >= 1)
                def _drain_prev():
                    wait_scatter(lax.rem(t + SLOTS - 1, SLOTS))

                @pl.when(t + 3 < CPG)
                def _prefetch():
                    issue_gather(t + 3, lax.rem(t + 3, SLOTS))

                return carry

            lax.fori_loop(0, CPG, chunk_body, carry)
            wait_scatter(lax.rem(CPG - 1, SLOTS))
            return carry

        lax.fori_loop(0, GROUPS, group_body, 0)
        plsc.subcore_barrier()

        pltpu.async_copy(
            agg_sh.at[pl.ds(row0, RPT)], agg_out.at[c, pl.ds(row0, RPT)], bsem
        )
        pltpu.async_copy(
            cnt_sh.at[pl.ds(row0, RPT)], cnt_out.at[c, pl.ds(row0, RPT)], bsem
        )

        @pl.when(s == NS - 1)
        def _write_tail():
            pltpu.sync_copy(
                agg_sh.at[pl.ds(NS * RPT, TAIL)], agg_out.at[c, pl.ds(NS * RPT, TAIL)]
            )
            pltpu.sync_copy(
                cnt_sh.at[pl.ds(NS * RPT, TAIL)], cnt_out.at[c, pl.ds(NS * RPT, TAIL)]
            )

        pltpu.make_async_copy(
            agg_sh.at[pl.ds(row0, RPT)], agg_out.at[c, pl.ds(row0, RPT)], bsem
        ).wait()
        pltpu.make_async_copy(
            cnt_sh.at[pl.ds(row0, RPT)], cnt_out.at[c, pl.ds(row0, RPT)], bsem
        ).wait()

    return seg_sum(x, src_r, dst_r)


def _tc_body(agg_ref, cnt_ref, x_ref, wl_ref, wr_ref, out_ref):
    agg = agg_ref[0] + agg_ref[1]
    cnt = cnt_ref[0, 0, :] + cnt_ref[0, 1, :]
    mean = agg / jnp.maximum(cnt, 1.0)[:, None]
    dn = (((1,), (1,)), ((), ()))
    out_ref[...] = (
        lax.dot_general(mean, wl_ref[...], dn, preferred_element_type=jnp.float32)
        + lax.dot_general(x_ref[...], wr_ref[...], dn, preferred_element_type=jnp.float32)
    )


def _tc_combine(agg2, cnt2, x, W_l, W_r):
    B = 1000
    grid = (N // B,)
    return pl.pallas_call(
        _tc_body,
        grid=grid,
        in_specs=[
            pl.BlockSpec((NC, B, D), lambda i: (0, i, 0)),
            pl.BlockSpec((1, NC, B), lambda i: (i, 0, 0)),
            pl.BlockSpec((B, D), lambda i: (i, 0)),
            pl.BlockSpec((D, D), lambda i: (0, 0)),
            pl.BlockSpec((D, D), lambda i: (0, 0)),
        ],
        out_specs=pl.BlockSpec((B, D), lambda i: (i, 0)),
        out_shape=jax.ShapeDtypeStruct((N, D), jnp.float32),
    )(agg2, cnt2.reshape(NC, N // B, B).transpose(1, 0, 2), x, W_l, W_r)


def kernel(x, edge_index, W_l, W_r):
    src_r = edge_index[0].reshape(NW, GROUPS, CPG, K)
    dst_r = edge_index[1].reshape(NW, GROUPS, CPG, K)
    agg2, cnt2 = _sc_segment_sum(x, src_r, dst_r)
    return _tc_combine(agg2, cnt2, x, W_l, W_r)
